# Initial kernel scaffold; baseline (speedup 1.0000x reference)
import jax
import jax.numpy as jnp
from jax.experimental import pallas as pl


def kernel(segment_ids, table):
    raise NotImplementedError("write your pallas kernel here")


# speedup vs baseline: 2.7726x; 2.7726x over previous
import functools

import jax
import jax.numpy as jnp
from jax import lax
from jax.experimental import pallas as pl
from jax.experimental.pallas import tpu as pltpu
from jax.experimental.pallas import tpu_sc as plsc

_BATCH = 4096
_SEQ_LEN = 200
_EMBED_DIM = 64

_NUM_CORES = 2
_NUM_SUBCORES = 16
_NW = _NUM_CORES * _NUM_SUBCORES

_TOKENS = _BATCH * _SEQ_LEN
_PER_W = _TOKENS // _NW
_IDXROW = 128
_CHUNK = 1024
_ROWS_PER_CHUNK = _CHUNK // _IDXROW
_NCHUNKS = _PER_W // _CHUNK

_mesh = plsc.VectorSubcoreMesh(core_axis_name="c", subcore_axis_name="s")


@functools.partial(
    pl.kernel,
    mesh=_mesh,
    out_type=jax.ShapeDtypeStruct((_TOKENS, _EMBED_DIM), jnp.float32),
    scratch_types=[
        pltpu.VMEM((_ROWS_PER_CHUNK, _IDXROW), jnp.int32),
        pltpu.VMEM((_CHUNK, _EMBED_DIM), jnp.float32),
        pltpu.SemaphoreType.DMA,
    ],
    compiler_params=pltpu.CompilerParams(use_tc_tiling_on_sc=False),
)
def _emb_lookup(ids_hbm, table_hbm, out_hbm, idx_v, rows_v, sem):
    wid = lax.axis_index("s") * _NUM_CORES + lax.axis_index("c")
    tok_base = wid * _PER_W
    row_base = tok_base // _IDXROW

    def chunk_body(i, carry):
        tok_off = pl.multiple_of(tok_base + i * _CHUNK, _CHUNK)
        row_off = pl.multiple_of(row_base + i * _ROWS_PER_CHUNK, _ROWS_PER_CHUNK)
        pltpu.sync_copy(ids_hbm.at[pl.ds(row_off, _ROWS_PER_CHUNK)], idx_v)
        copies = []
        for j in range(_ROWS_PER_CHUNK):
            copies.append(
                pltpu.async_copy(
                    table_hbm.at[idx_v.at[j]],
                    rows_v.at[pl.ds(j * _IDXROW, _IDXROW)],
                    sem,
                )
            )
        for cp in copies:
            cp.wait()
        pltpu.sync_copy(rows_v, out_hbm.at[pl.ds(tok_off, _CHUNK)])
        return carry

    lax.fori_loop(0, _NCHUNKS, chunk_body, 0)


def kernel(segment_ids, table):
    ids2d = segment_ids.reshape(_TOKENS // _IDXROW, _IDXROW)
    out = _emb_lookup(ids2d, table)
    return out.reshape(_BATCH, _SEQ_LEN, _EMBED_DIM)


# speedup vs baseline: 2.7832x; 1.0038x over previous
import functools

import jax
import jax.numpy as jnp
from jax import lax
from jax.experimental import pallas as pl
from jax.experimental.pallas import tpu as pltpu
from jax.experimental.pallas import tpu_sc as plsc

_BATCH = 4096
_SEQ_LEN = 200
_EMBED_DIM = 64

_NUM_CORES = 2
_NUM_SUBCORES = 16
_NW = _NUM_CORES * _NUM_SUBCORES

_TOKENS = _BATCH * _SEQ_LEN
_PER_W = _TOKENS // _NW
_IDXROW = 128
_CHUNK = 512
_RPC = _CHUNK // _IDXROW
_NCH = _PER_W // _CHUNK
_NPAIR = _NCH // 2

_mesh = plsc.VectorSubcoreMesh(core_axis_name="c", subcore_axis_name="s")


@functools.partial(
    pl.kernel,
    mesh=_mesh,
    out_type=jax.ShapeDtypeStruct((_TOKENS, _EMBED_DIM), jnp.float32),
    scratch_types=[
        pltpu.VMEM((_RPC, _IDXROW), jnp.int32),
        pltpu.VMEM((_RPC, _IDXROW), jnp.int32),
        pltpu.VMEM((_CHUNK, _EMBED_DIM), jnp.float32),
        pltpu.VMEM((_CHUNK, _EMBED_DIM), jnp.float32),
        pltpu.SemaphoreType.DMA,
        pltpu.SemaphoreType.DMA,
        pltpu.SemaphoreType.DMA,
        pltpu.SemaphoreType.DMA,
    ],
    compiler_params=pltpu.CompilerParams(use_tc_tiling_on_sc=False),
)
def _emb_lookup(ids_hbm, table_hbm, out_hbm, idx0, idx1, rows0, rows1,
                gsem0, gsem1, wsem0, wsem1):
    wid = lax.axis_index("s") * _NUM_CORES + lax.axis_index("c")
    tok_base = wid * _PER_W
    row_base = tok_base // _IDXROW

    def tok_off(i):
        return pl.multiple_of(tok_base + i * _CHUNK, _CHUNK)

    def row_off(i):
        return pl.multiple_of(row_base + i * _RPC, _RPC)

    def fire_gather(i, idxb, rowsb, gsem):
        pltpu.sync_copy(ids_hbm.at[pl.ds(row_off(i), _RPC)], idxb)
        for j in range(_RPC):
            pltpu.async_copy(
                table_hbm.at[idxb.at[j]],
                rowsb.at[pl.ds(j * _IDXROW, _IDXROW)],
                gsem,
            )

    def wait_gather(idxb, rowsb, gsem):
        for j in range(_RPC):
            pltpu.make_async_copy(
                table_hbm.at[idxb.at[j]],
                rowsb.at[pl.ds(j * _IDXROW, _IDXROW)],
                gsem,
            ).wait()

    def fire_write(i, rowsb, wsem):
        pltpu.async_copy(rowsb, out_hbm.at[pl.ds(tok_off(i), _CHUNK)], wsem)

    def wait_write(rowsb, wsem):
        pltpu.make_async_copy(
            rowsb, out_hbm.at[pl.ds(tok_base, _CHUNK)], wsem
        ).wait()

    fire_gather(0, idx0, rows0, gsem0)

    def pair_body(k, carry):
        i = 2 * k

        @pl.when(k > 0)
        def _():
            wait_write(rows1, wsem1)

        fire_gather(i + 1, idx1, rows1, gsem1)
        wait_gather(idx0, rows0, gsem0)
        fire_write(i, rows0, wsem0)

        @pl.when(k < _NPAIR - 1)
        def _():
            wait_write(rows0, wsem0)
            fire_gather(i + 2, idx0, rows0, gsem0)

        wait_gather(idx1, rows1, gsem1)
        fire_write(i + 1, rows1, wsem1)
        return carry

    lax.fori_loop(0, _NPAIR, pair_body, 0)
    wait_write(rows0, wsem0)
    wait_write(rows1, wsem1)


def kernel(segment_ids, table):
    ids2d = segment_ids.reshape(_TOKENS // _IDXROW, _IDXROW)
    out = _emb_lookup(ids2d, table)
    return out.reshape(_BATCH, _SEQ_LEN, _EMBED_DIM)


# speedup vs baseline: 2.7857x; 1.0009x over previous
import functools

import jax
import jax.numpy as jnp
from jax import lax
from jax.experimental import pallas as pl
from jax.experimental.pallas import tpu as pltpu
from jax.experimental.pallas import tpu_sc as plsc

_BATCH = 4096
_SEQ_LEN = 200
_EMBED_DIM = 64

_NUM_CORES = 2
_NUM_SUBCORES = 16
_NW = _NUM_CORES * _NUM_SUBCORES

_TOKENS = _BATCH * _SEQ_LEN
_PER_W = _TOKENS // _NW
_IDXROW = 128
_CHUNK = 512
_RPC = _CHUNK // _IDXROW
_NCH = _PER_W // _CHUNK
_NPAIR = _NCH // 2

_mesh = plsc.VectorSubcoreMesh(core_axis_name="c", subcore_axis_name="s")


@functools.partial(
    pl.kernel,
    mesh=_mesh,
    out_type=jax.ShapeDtypeStruct((_TOKENS, _EMBED_DIM), jnp.float32),
    scratch_types=[
        pltpu.VMEM((_CHUNK,), jnp.int32),
        pltpu.VMEM((_CHUNK,), jnp.int32),
        pltpu.VMEM((_CHUNK, _EMBED_DIM), jnp.float32),
        pltpu.VMEM((_CHUNK, _EMBED_DIM), jnp.float32),
        pltpu.SemaphoreType.DMA,
        pltpu.SemaphoreType.DMA,
        pltpu.SemaphoreType.DMA,
        pltpu.SemaphoreType.DMA,
    ],
    compiler_params=pltpu.CompilerParams(use_tc_tiling_on_sc=False),
)
def _emb_lookup(ids_hbm, table_hbm, out_hbm, idx0, idx1, rows0, rows1,
                gsem0, gsem1, wsem0, wsem1):
    wid = lax.axis_index("s") * _NUM_CORES + lax.axis_index("c")
    tok_base = wid * _PER_W

    def tok_off(i):
        return pl.multiple_of(tok_base + i * _CHUNK, _CHUNK)

    def fire_gather(i, idxb, rowsb, gsem):
        pltpu.sync_copy(ids_hbm.at[pl.ds(tok_off(i), _CHUNK)], idxb)
        pltpu.async_copy(table_hbm.at[idxb], rowsb, gsem)

    def wait_gather(idxb, rowsb, gsem):
        pltpu.make_async_copy(table_hbm.at[idxb], rowsb, gsem).wait()

    def fire_write(i, rowsb, wsem):
        pltpu.async_copy(rowsb, out_hbm.at[pl.ds(tok_off(i), _CHUNK)], wsem)

    def wait_write(rowsb, wsem):
        pltpu.make_async_copy(
            rowsb, out_hbm.at[pl.ds(tok_base, _CHUNK)], wsem
        ).wait()

    fire_gather(0, idx0, rows0, gsem0)

    def pair_body(k, carry):
        i = 2 * k

        @pl.when(k > 0)
        def _():
            wait_write(rows1, wsem1)

        fire_gather(i + 1, idx1, rows1, gsem1)
        wait_gather(idx0, rows0, gsem0)
        fire_write(i, rows0, wsem0)

        @pl.when(k < _NPAIR - 1)
        def _():
            wait_write(rows0, wsem0)
            fire_gather(i + 2, idx0, rows0, gsem0)

        wait_gather(idx1, rows1, gsem1)
        fire_write(i + 1, rows1, wsem1)
        return carry

    lax.fori_loop(0, _NPAIR, pair_body, 0)
    wait_write(rows0, wsem0)
    wait_write(rows1, wsem1)


def kernel(segment_ids, table):
    ids_flat = segment_ids.reshape(_TOKENS)
    out = _emb_lookup(ids_flat, table)
    return out.reshape(_BATCH, _SEQ_LEN, _EMBED_DIM)


# speedup vs baseline: 3.9319x; 1.4115x over previous
import functools

import jax
import jax.numpy as jnp
from jax import lax
from jax.experimental import pallas as pl
from jax.experimental.pallas import tpu as pltpu
from jax.experimental.pallas import tpu_sc as plsc

_BATCH = 4096
_SEQ_LEN = 200
_EMBED_DIM = 64

_NUM_CORES = 2
_NUM_SUBCORES = 16
_NW = _NUM_CORES * _NUM_SUBCORES
_LB = _BATCH // _NW
_ET = _EMBED_DIM // 8
_L = 16
_IDS_PER_W = _LB * _SEQ_LEN
_TBL = _SEQ_LEN * _EMBED_DIM
_TBL_SL = _TBL - (_EMBED_DIM - 8)

_mesh = plsc.VectorSubcoreMesh(core_axis_name="c", subcore_axis_name="s")


@functools.partial(
    pl.kernel,
    mesh=_mesh,
    out_type=jax.ShapeDtypeStruct((_SEQ_LEN, _ET, _NW, 1024), jnp.float32),
    scratch_types=[
        pltpu.VMEM((_IDS_PER_W,), jnp.int32),
        pltpu.VMEM((_IDS_PER_W,), jnp.int32),
        pltpu.VMEM((_TBL,), jnp.float32),
        pltpu.VMEM((_ET, 1024), jnp.float32),
        pltpu.VMEM((_ET, 1024), jnp.float32),
        pltpu.SemaphoreType.DMA,
        pltpu.SemaphoreType.DMA,
    ],
    compiler_params=pltpu.CompilerParams(
        use_tc_tiling_on_sc=False, needs_layout_passes=False
    ),
)
def _emb_lookup(ids_hbm, table_hbm, out_hbm, ids_v, idsT_v, table_v,
                buf0, buf1, wsem0, wsem1):
    wid = lax.axis_index("s") * _NUM_CORES + lax.axis_index("c")
    pltpu.sync_copy(ids_hbm.at[pl.ds(wid * _IDS_PER_W, _IDS_PER_W)], ids_v)
    pltpu.sync_copy(table_hbm, table_v)

    iota_s = lax.iota(jnp.int32, _L) * _SEQ_LEN

    def transpose_body(s, carry):
        gathered = [
            plsc.load_gather(ids_v, [iota_s + (g * _L * _SEQ_LEN + s)])
            for g in range(_LB // _L)
        ]
        for g in range(_LB // _L):
            idsT_v[pl.ds(s * _LB + g * _L, _L)] = gathered[g] * _EMBED_DIM
        return carry

    lax.fori_loop(0, _SEQ_LEN, transpose_body, 0)

    def unit(s, bufb):
        for g in range(_LB // _L):
            ids64 = idsT_v[pl.ds(s * _LB + g * _L, _L)]
            ids64_ei = [ids64 + ei for ei in range(8)]
            for et in range(_ET):
                tbl_et = table_v.at[pl.ds(et * 8, _TBL_SL)]
                vals = [
                    plsc.load_gather(tbl_et, [ids64_ei[ei]]) for ei in range(8)
                ]
                for ei in range(8):
                    bufb[et, pl.ds(ei * 128 + g * _L, _L)] = vals[ei]

    def fire_write(s, bufb, wsem):
        pltpu.async_copy(bufb, out_hbm.at[s, :, wid], wsem)

    def wait_write(bufb, wsem):
        pltpu.make_async_copy(bufb, out_hbm.at[0, :, wid], wsem).wait()

    def pair_body(k, carry):
        s0 = 2 * k

        @pl.when(k > 0)
        def _():
            wait_write(buf0, wsem0)

        unit(s0, buf0)
        fire_write(s0, buf0, wsem0)

        @pl.when(k > 0)
        def _():
            wait_write(buf1, wsem1)

        unit(s0 + 1, buf1)
        fire_write(s0 + 1, buf1, wsem1)
        return carry

    lax.fori_loop(0, _SEQ_LEN // 2, pair_body, 0)
    wait_write(buf0, wsem0)
    wait_write(buf1, wsem1)


def kernel(segment_ids, table):
    ids_flat = segment_ids.reshape(_BATCH * _SEQ_LEN)
    table_flat = table.reshape(_TBL)
    out5 = _emb_lookup(ids_flat, table_flat)
    out = out5.reshape(_SEQ_LEN, _ET, _NW, 8, 128)
    out = out.transpose(2, 4, 0, 1, 3)
    return out.reshape(_BATCH, _SEQ_LEN, _EMBED_DIM)


# speedup vs baseline: 9.2855x; 2.3616x over previous
import functools

import jax
import jax.numpy as jnp
from jax import lax
from jax.experimental import pallas as pl
from jax.experimental.pallas import tpu as pltpu
from jax.experimental.pallas import tpu_sc as plsc

_BATCH = 4096
_SEQ_LEN = 200
_EMBED_DIM = 64

_NUM_CORES = 2
_NUM_SUBCORES = 16
_NW = _NUM_CORES * _NUM_SUBCORES
_LB = _BATCH // _NW
_ET = _EMBED_DIM // 8
_L = 16
_IDS_PER_W = _LB * _SEQ_LEN
_TSTRIDE = _EMBED_DIM + 1
_TBL = _SEQ_LEN * _TSTRIDE
_TBL_SL = _TBL - (_EMBED_DIM - 8)

_mesh = plsc.VectorSubcoreMesh(core_axis_name="c", subcore_axis_name="s")


@functools.partial(
    pl.kernel,
    mesh=_mesh,
    out_type=jax.ShapeDtypeStruct((_SEQ_LEN, _ET, _NW, 1024), jnp.float32),
    scratch_types=[
        pltpu.VMEM((_IDS_PER_W,), jnp.int32),
        pltpu.VMEM((_IDS_PER_W,), jnp.int32),
        pltpu.VMEM((_TBL,), jnp.float32),
        pltpu.VMEM((_ET, 1024), jnp.float32),
        pltpu.VMEM((_ET, 1024), jnp.float32),
        pltpu.SemaphoreType.DMA,
        pltpu.SemaphoreType.DMA,
    ],
    compiler_params=pltpu.CompilerParams(
        use_tc_tiling_on_sc=False, needs_layout_passes=False
    ),
)
def _emb_lookup(ids_hbm, table_hbm, out_hbm, ids_v, idsT_v, table_v,
                buf0, buf1, wsem0, wsem1):
    wid = lax.axis_index("s") * _NUM_CORES + lax.axis_index("c")
    pltpu.sync_copy(ids_hbm.at[pl.ds(wid * _IDS_PER_W, _IDS_PER_W)], ids_v)
    pltpu.sync_copy(table_hbm, table_v)

    iota_s = lax.iota(jnp.int32, _L) * _SEQ_LEN

    def transpose_body(s, carry):
        gathered = [
            plsc.load_gather(ids_v, [iota_s + (g * _L * _SEQ_LEN + s)])
            for g in range(_LB // _L)
        ]
        for g in range(_LB // _L):
            idsT_v[pl.ds(s * _LB + g * _L, _L)] = gathered[g] * _TSTRIDE
        return carry

    lax.fori_loop(0, _SEQ_LEN, transpose_body, 0)

    def unit(s, bufb):
        for g in range(_LB // _L):
            ids64 = idsT_v[pl.ds(s * _LB + g * _L, _L)]
            ids64_ei = [ids64 + ei for ei in range(8)]
            for et in range(_ET):
                tbl_et = table_v.at[pl.ds(et * 8, _TBL_SL)]
                vals = [
                    plsc.load_gather(tbl_et, [ids64_ei[ei]]) for ei in range(8)
                ]
                for ei in range(8):
                    bufb[et, pl.ds(ei * 128 + g * _L, _L)] = vals[ei]

    def fire_write(s, bufb, wsem):
        pltpu.async_copy(bufb, out_hbm.at[s, :, wid], wsem)

    def wait_write(bufb, wsem):
        pltpu.make_async_copy(bufb, out_hbm.at[0, :, wid], wsem).wait()

    def pair_body(k, carry):
        s0 = 2 * k

        @pl.when(k > 0)
        def _():
            wait_write(buf0, wsem0)

        unit(s0, buf0)
        fire_write(s0, buf0, wsem0)

        @pl.when(k > 0)
        def _():
            wait_write(buf1, wsem1)

        unit(s0 + 1, buf1)
        fire_write(s0 + 1, buf1, wsem1)
        return carry

    lax.fori_loop(0, _SEQ_LEN // 2, pair_body, 0)
    wait_write(buf0, wsem0)
    wait_write(buf1, wsem1)


def kernel(segment_ids, table):
    ids_flat = segment_ids.reshape(_BATCH * _SEQ_LEN)
    table_flat = jnp.pad(table, ((0, 0), (0, 1))).reshape(_TBL)
    out5 = _emb_lookup(ids_flat, table_flat)
    out = out5.reshape(_SEQ_LEN, _ET, _NW, 8, 128)
    out = out.transpose(2, 4, 0, 1, 3)
    return out.reshape(_BATCH, _SEQ_LEN, _EMBED_DIM)
